# baseline (speedup 1.0000x reference)
import functools

import jax
import jax.numpy as jnp
from jax import lax
from jax.experimental import pallas as pl
from jax.experimental.pallas import tpu as pltpu
from jax.experimental.pallas import tpu_sc as plsc

B = 320000
N_ROWS = 10000
D = 128
NW = 32
E_PER_W = B // NW
C = 80
N_CHUNKS = E_PER_W // C
GROUPS = C // 16

_mesh = plsc.VectorSubcoreMesh(core_axis_name="c", subcore_axis_name="s")


@functools.partial(
    pl.kernel,
    out_type=jax.ShapeDtypeStruct((B,), jnp.float32),
    mesh=_mesh,
    scratch_types=[
        pltpu.VMEM((2, 2, C), jnp.int32),
        pltpu.VMEM((2, C), jnp.float32),
        pltpu.VMEM((2, C, D), jnp.float32),
        pltpu.VMEM((2, C, D), jnp.float32),
        pltpu.VMEM((16, 17), jnp.float32),
        pltpu.VMEM_SHARED((N_ROWS, D), jnp.float32),
        pltpu.SemaphoreType.DMA,
        pltpu.SemaphoreType.DMA,
        pltpu.SemaphoreType.DMA,
        pltpu.SemaphoreType.DMA,
    ],
    compiler_params=pltpu.CompilerParams(needs_layout_passes=False),
)
def _sc_kernel(x_sotu_hbm, x_taxon_hbm, idx0_hbm, idx1_hbm, out_hbm,
               idx_c, out_c, rows0_v, rows1_v, tr_v, sotu_sp,
               sem_i, sem_a, sem_b, sem_o):
    sid = lax.axis_index("s")
    wid = sid * 2 + lax.axis_index("c")
    base_w = wid * E_PER_W
    lane = lax.iota(jnp.int32, 16)

    bounce = rows1_v.at[0]

    def stage_block(s, _):
        off = sid * 624 + s * 80
        pltpu.sync_copy(x_sotu_hbm.at[pl.ds(off, 80)], bounce)
        pltpu.sync_copy(bounce, sotu_sp.at[pl.ds(off, 80)])
        return 0

    lax.fori_loop(0, 7, stage_block, 0)
    off64 = sid * 624 + 560
    pltpu.sync_copy(x_sotu_hbm.at[pl.ds(off64, 64)], bounce.at[pl.ds(0, 64)])
    pltpu.sync_copy(bounce.at[pl.ds(0, 64)], sotu_sp.at[pl.ds(off64, 64)])

    @pl.when(sid == 0)
    def _stage_tail():
        pltpu.sync_copy(x_sotu_hbm.at[pl.ds(9984, 16)],
                        bounce.at[pl.ds(0, 16)])
        pltpu.sync_copy(bounce.at[pl.ds(0, 16)], sotu_sp.at[pl.ds(9984, 16)])

    plsc.subcore_barrier()

    def fire_idx(it):
        p = jnp.bitwise_and(it, 1)
        pltpu.async_copy(
            idx0_hbm.at[pl.ds(base_w + it * C, C)], idx_c.at[p, 0], sem_i)
        pltpu.async_copy(
            idx1_hbm.at[pl.ds(base_w + it * C, C)], idx_c.at[p, 1], sem_i)

    def drain_idx(it):
        p = jnp.bitwise_and(it, 1)
        pltpu.make_async_copy(
            idx0_hbm.at[pl.ds(base_w + it * C, C)], idx_c.at[p, 0],
            sem_i).wait()
        pltpu.make_async_copy(
            idx1_hbm.at[pl.ds(base_w + it * C, C)], idx_c.at[p, 1],
            sem_i).wait()

    def fire_rows(it):
        p = jnp.bitwise_and(it, 1)
        pltpu.async_copy(sotu_sp.at[idx_c.at[p, 0]], rows0_v.at[p], sem_a)
        pltpu.async_copy(
            x_taxon_hbm.at[idx_c.at[p, 1]], rows1_v.at[p], sem_b)

    def drain_rows(it):
        p = jnp.bitwise_and(it, 1)
        pltpu.make_async_copy(
            sotu_sp.at[idx_c.at[p, 0]], rows0_v.at[p], sem_a).wait()
        pltpu.make_async_copy(
            x_taxon_hbm.at[idx_c.at[p, 1]], rows1_v.at[p], sem_b).wait()

    def fire_out(it):
        p = jnp.bitwise_and(it, 1)
        pltpu.async_copy(
            out_c.at[p], out_hbm.at[pl.ds(base_w + it * C, C)], sem_o)

    def drain_out(it):
        p = jnp.bitwise_and(it, 1)
        pltpu.make_async_copy(
            out_c.at[p], out_hbm.at[pl.ds(base_w + it * C, C)],
            sem_o).wait()

    def compute(it):
        p = jnp.bitwise_and(it, 1)
        r0 = rows0_v.at[p]
        r1 = rows1_v.at[p]

        def group_body(g, _):
            gbase = g * 16
            for e in range(16):
                row = gbase + e
                ps = []
                for k in range(8):
                    a = r0[row, pl.ds(16 * k, 16)]
                    b = r1[row, pl.ds(16 * k, 16)]
                    ps.append(a * b)
                s01 = ps[0] + ps[1]
                s23 = ps[2] + ps[3]
                s45 = ps[4] + ps[5]
                s67 = ps[6] + ps[7]
                tr_v[e, pl.ds(0, 16)] = (s01 + s23) + (s45 + s67)
            one = jnp.ones((16,), jnp.int32)
            col = jnp.zeros((16,), jnp.int32)
            acc = jnp.zeros((16,), jnp.float32)
            for c in range(16):
                acc = acc + plsc.load_gather(tr_v, [lane, col])
                col = col + one
            out_c[p, pl.ds(gbase, 16)] = acc
            return 0

        lax.fori_loop(0, GROUPS, group_body, 0)

    fire_idx(0)
    drain_idx(0)
    fire_rows(0)
    fire_idx(1)

    def body(it, _):
        drain_idx(it + 1)
        drain_rows(it)
        fire_rows(it + 1)

        @pl.when(it + 2 < N_CHUNKS)
        def _():
            fire_idx(it + 2)

        @pl.when(it > 0)
        def _():
            drain_out(it - 1)

        compute(it)
        fire_out(it)
        return 0

    lax.fori_loop(0, N_CHUNKS - 1, body, 0)
    drain_rows(N_CHUNKS - 1)
    drain_out(N_CHUNKS - 2)
    compute(N_CHUNKS - 1)
    fire_out(N_CHUNKS - 1)
    drain_out(N_CHUNKS - 1)


def kernel(x_sotu, x_taxon, edge_label_index):
    return _sc_kernel(x_sotu, x_taxon,
                      edge_label_index[0], edge_label_index[1])


# speedup vs baseline: 1.0528x; 1.0528x over previous
import functools

import jax
import jax.numpy as jnp
from jax import lax
from jax.experimental import pallas as pl
from jax.experimental.pallas import tpu as pltpu
from jax.experimental.pallas import tpu_sc as plsc

B = 320000
N_ROWS = 10000
D = 128
NW = 32
E_PER_W = B // NW
C = 80
N_CHUNKS = E_PER_W // C
GROUPS = C // 16

_mesh = plsc.VectorSubcoreMesh(core_axis_name="c", subcore_axis_name="s")


@functools.partial(
    pl.kernel,
    out_type=jax.ShapeDtypeStruct((B,), jnp.float32),
    mesh=_mesh,
    scratch_types=[
        pltpu.VMEM((E_PER_W,), jnp.int32),
        pltpu.VMEM((E_PER_W,), jnp.int32),
        pltpu.VMEM((E_PER_W,), jnp.float32),
        pltpu.VMEM((2, C, D), jnp.float32),
        pltpu.VMEM((2, C, D), jnp.float32),
        pltpu.VMEM((C, 17), jnp.float32),
        pltpu.SemaphoreType.DMA,
        pltpu.SemaphoreType.DMA,
    ],
    compiler_params=pltpu.CompilerParams(needs_layout_passes=False),
)
def _sc_kernel(x_sotu_hbm, x_taxon_hbm, idx0_hbm, idx1_hbm, out_hbm,
               idx0_v, idx1_v, out_v, rows0_v, rows1_v,
               tr_v, sem_a, sem_b):
    wid = lax.axis_index("s") * 2 + lax.axis_index("c")
    base_w = wid * E_PER_W
    lane = lax.iota(jnp.int32, 16)

    pltpu.sync_copy(idx0_hbm.at[pl.ds(base_w, E_PER_W)], idx0_v)
    pltpu.sync_copy(idx1_hbm.at[pl.ds(base_w, E_PER_W)], idx1_v)

    def fire(it):
        p = jnp.bitwise_and(it, 1)
        pltpu.async_copy(
            x_sotu_hbm.at[idx0_v.at[pl.ds(it * C, C)]], rows0_v.at[p], sem_a)
        pltpu.async_copy(
            x_taxon_hbm.at[idx1_v.at[pl.ds(it * C, C)]], rows1_v.at[p], sem_b)

    def drain(it):
        p = jnp.bitwise_and(it, 1)
        pltpu.make_async_copy(
            x_sotu_hbm.at[idx0_v.at[pl.ds(it * C, C)]], rows0_v.at[p],
            sem_a).wait()
        pltpu.make_async_copy(
            x_taxon_hbm.at[idx1_v.at[pl.ds(it * C, C)]], rows1_v.at[p],
            sem_b).wait()

    def compute(it):
        p = jnp.bitwise_and(it, 1)
        r0 = rows0_v.at[p]
        r1 = rows1_v.at[p]

        def group_body(g, _):
            gbase = g * 16
            for e in range(16):
                row = gbase + e
                ps = []
                for k in range(8):
                    a = r0[row, pl.ds(16 * k, 16)]
                    b = r1[row, pl.ds(16 * k, 16)]
                    ps.append(a * b)
                s01 = ps[0] + ps[1]
                s23 = ps[2] + ps[3]
                s45 = ps[4] + ps[5]
                s67 = ps[6] + ps[7]
                tr_v[row, pl.ds(0, 16)] = (s01 + s23) + (s45 + s67)
            one = jnp.ones((16,), jnp.int32)
            col = jnp.zeros((16,), jnp.int32)
            acc = jnp.zeros((16,), jnp.float32)
            row_idx = lane + gbase
            for c in range(16):
                acc = acc + plsc.load_gather(tr_v, [row_idx, col])
                col = col + one
            out_v[pl.ds(it * C + gbase, 16)] = acc
            return 0

        lax.fori_loop(0, GROUPS, group_body, 0)

    fire(0)

    def body(it, _):
        drain(it)
        fire(it + 1)
        compute(it)
        return 0

    lax.fori_loop(0, N_CHUNKS - 1, body, 0)
    drain(N_CHUNKS - 1)
    compute(N_CHUNKS - 1)

    pltpu.sync_copy(out_v, out_hbm.at[pl.ds(base_w, E_PER_W)])


def kernel(x_sotu, x_taxon, edge_label_index):
    return _sc_kernel(x_sotu, x_taxon,
                      edge_label_index[0], edge_label_index[1])
